# baseline (speedup 1.0000x reference)
import dataclasses
import functools

import jax
import jax.numpy as jnp
from jax import lax
from jax.experimental import pallas as pl
from jax.experimental.pallas import tpu as pltpu
from jax.experimental.pallas import tpu_sc as plsc

_N = 10000
_E = 160000
_D = 256
_HALF = 128
_LANES = 16

_TILES = 16
_CHUNK = 128
_CPT = 80
_PCPT = _CPT // 2
_EPT = _CPT * _CHUNK
_EPAD = _EPT * _TILES

_ROWS_PER_TILE = 632
_LAST_START = _N - _ROWS_PER_TILE


def _mm1_body(x_ref, wa_ref, wb_ref, o0_ref, o1_ref):
    xx = x_ref[...]
    o0_ref[...] = jnp.dot(xx, wa_ref[...], preferred_element_type=jnp.float32)
    o1_ref[...] = jnp.dot(xx, wb_ref[...], preferred_element_type=jnp.float32)


_matmul1 = pl.pallas_call(
    _mm1_body,
    grid=(10,),
    in_specs=[
        pl.BlockSpec((1000, _D), lambda i: (i, 0)),
        pl.BlockSpec((_D, _HALF), lambda i: (0, 0)),
        pl.BlockSpec((_D, _HALF), lambda i: (0, 0)),
    ],
    out_specs=[
        pl.BlockSpec((1000, _HALF), lambda i: (i, 0)),
        pl.BlockSpec((1000, _HALF), lambda i: (i, 0)),
    ],
    out_shape=[jax.ShapeDtypeStruct((_N, _HALF), jnp.float32)] * 2,
)


def _mm2_body(a_ref, b_ref, w2_ref, o0_ref, o1_ref):
    a = jnp.maximum(a_ref[...], 0.0)
    b = jnp.maximum(b_ref[...], 0.0)
    w = w2_ref[...]
    o0_ref[...] = (
        jnp.dot(a, w[:_HALF, :_HALF], preferred_element_type=jnp.float32)
        + jnp.dot(b, w[_HALF:, :_HALF], preferred_element_type=jnp.float32)
    )
    o1_ref[...] = (
        jnp.dot(a, w[:_HALF, _HALF:], preferred_element_type=jnp.float32)
        + jnp.dot(b, w[_HALF:, _HALF:], preferred_element_type=jnp.float32)
    )


_matmul2 = pl.pallas_call(
    _mm2_body,
    grid=(10,),
    in_specs=[
        pl.BlockSpec((1000, _HALF), lambda i: (i, 0)),
        pl.BlockSpec((1000, _HALF), lambda i: (i, 0)),
        pl.BlockSpec((_D, _D), lambda i: (0, 0)),
    ],
    out_specs=[
        pl.BlockSpec((1000, _HALF), lambda i: (i, 0)),
        pl.BlockSpec((1000, _HALF), lambda i: (i, 0)),
    ],
    out_shape=[jax.ShapeDtypeStruct((_N, _HALF), jnp.float32)] * 2,
)


def _spmm_body(src_hbm, dst_hbm, w_hbm, h0_hbm, h1_hbm, init0_hbm, init1_hbm,
               out0_hbm, out1_hbm, acc, gidx, sidx, wbuf, rows0, rows1,
               semg0, semg1, sems0, sems1):
    c = lax.axis_index("c")
    s = lax.axis_index("s")
    row0 = jnp.where(s == _TILES - 1, _LAST_START, s * _ROWS_PER_TILE)
    rsl = pl.ds(row0, _ROWS_PER_TILE)

    @pl.when(c == 0)
    def _():
        pltpu.sync_copy(init0_hbm.at[rsl], acc.at[rsl])

    @pl.when(c == 1)
    def _():
        pltpu.sync_copy(init1_hbm.at[rsl], acc.at[rsl])

    plsc.subcore_barrier()

    h_hbm = (h0_hbm, h1_hbm)
    rows = (rows0, rows1)
    semg = (semg0, semg1)
    sems = (sems0, sems1)

    def _gather(k, b):
        @pl.when(c == 0)
        def _():
            pltpu.async_copy(h_hbm[0].at[gidx.at[k]], rows[b], semg[b])

        @pl.when(c == 1)
        def _():
            pltpu.async_copy(h_hbm[1].at[gidx.at[k]], rows[b], semg[b])

    def _gather_wait(k, b):
        @pl.when(c == 0)
        def _():
            pltpu.make_async_copy(h_hbm[0].at[gidx.at[k]], rows[b],
                                  semg[b]).wait()

        @pl.when(c == 1)
        def _():
            pltpu.make_async_copy(h_hbm[1].at[gidx.at[k]], rows[b],
                                  semg[b]).wait()

    def _scale(k, b):
        @plsc.parallel_loop(0, _CHUNK, unroll=4)
        def _(e):
            wv = plsc.load_gather(
                wbuf, [jnp.full((_LANES,), k, jnp.int32),
                       jnp.full((_LANES,), e, jnp.int32)])
            for j in range(_HALF // _LANES):
                sl = (e, pl.ds(j * _LANES, _LANES))
                rows[b][sl] = rows[b][sl] * wv

    @pl.loop(0, 2)
    def _(p):
        psl = pl.ds(p * _PCPT, _PCPT)
        pltpu.sync_copy(src_hbm.at[s, psl], gidx)
        pltpu.sync_copy(dst_hbm.at[s, psl], sidx)
        pltpu.sync_copy(w_hbm.at[s, psl], wbuf)

        _gather(0, 0)
        _gather(1, 1)

        @pl.loop(0, _PCPT, step=2)
        def _(k):
            _gather_wait(k, 0)
            _scale(k, 0)
            d0 = pltpu.async_copy(rows0, acc.at[sidx.at[k]], sems0, add=True)
            _gather_wait(k + 1, 1)
            _scale(k + 1, 1)
            d1 = pltpu.async_copy(rows1, acc.at[sidx.at[k + 1]], sems1,
                                  add=True)
            d0.wait()

            @pl.when(k + 2 < _PCPT)
            def _():
                _gather(k + 2, 0)

            d1.wait()

            @pl.when(k + 3 < _PCPT)
            def _():
                _gather(k + 3, 1)

    plsc.subcore_barrier()

    @pl.when(c == 0)
    def _():
        pltpu.sync_copy(acc.at[rsl], out0_hbm.at[rsl])

    @pl.when(c == 1)
    def _():
        pltpu.sync_copy(acc.at[rsl], out1_hbm.at[rsl])


_sc_params = pltpu.CompilerParams()
if "needs_layout_passes" in pltpu.CompilerParams.__dataclass_fields__:
    _sc_params = dataclasses.replace(_sc_params, needs_layout_passes=False)

_spmm = pl.kernel(
    _spmm_body,
    compiler_params=_sc_params,
    out_type=(
        jax.ShapeDtypeStruct((_N, _HALF), jnp.float32),
        jax.ShapeDtypeStruct((_N, _HALF), jnp.float32),
    ),
    mesh=plsc.VectorSubcoreMesh(
        core_axis_name="c", subcore_axis_name="s", num_cores=2, num_subcores=16
    ),
    scratch_types=[
        pltpu.VMEM_SHARED((_N, _HALF), jnp.float32),
        pltpu.VMEM((_PCPT, _CHUNK), jnp.int32),
        pltpu.VMEM((_PCPT, _CHUNK), jnp.int32),
        pltpu.VMEM((_PCPT, _CHUNK), jnp.float32),
        pltpu.VMEM((_CHUNK, _HALF), jnp.float32),
        pltpu.VMEM((_CHUNK, _HALF), jnp.float32),
        pltpu.SemaphoreType.DMA,
        pltpu.SemaphoreType.DMA,
        pltpu.SemaphoreType.DMA,
        pltpu.SemaphoreType.DMA,
    ],
)


def kernel(x, edge_index, edge_weight, w1, b1, w2, b2):
    order = jnp.argsort(edge_index[1])
    dst = edge_index[0][order]
    src = edge_index[1][order]
    edge_weight = edge_weight[order]
    pad = _EPAD - _E
    shape3 = (_TILES, _CPT, _CHUNK)
    src_p = jnp.concatenate([src, jnp.zeros((pad,), jnp.int32)]).reshape(shape3)
    dst_p = jnp.concatenate([dst, jnp.zeros((pad,), jnp.int32)]).reshape(shape3)
    w_p = jnp.concatenate(
        [edge_weight, jnp.zeros((pad,), jnp.float32)]).reshape(shape3)

    h0, h1 = _matmul1(x, w1[:, :_HALF], w1[:, _HALF:])
    b1_0 = jnp.broadcast_to(b1[None, :_HALF], (_N, _HALF))
    b1_1 = jnp.broadcast_to(b1[None, _HALF:], (_N, _HALF))
    s1_0, s1_1 = _spmm(src_p, dst_p, w_p, h0, h1, b1_0, b1_1)

    h2_0, h2_1 = _matmul2(s1_0, s1_1, w2)
    b2_0 = jnp.broadcast_to(b2[None, :_HALF], (_N, _HALF))
    b2_1 = jnp.broadcast_to(b2[None, _HALF:], (_N, _HALF))
    o0, o1 = _spmm(src_p, dst_p, w_p, h2_0, h2_1, b2_0, b2_1)

    return jnp.concatenate([o0, o1], axis=1)


# speedup vs baseline: 1.5774x; 1.5774x over previous
import dataclasses

import jax
import jax.numpy as jnp
from jax import lax
from jax.experimental import pallas as pl
from jax.experimental.pallas import tpu as pltpu
from jax.experimental.pallas import tpu_sc as plsc

_N = 10000
_E = 160000
_D = 256
_HALF = 128
_LANES = 16

_TILES = 16
_CHUNK = 64
_NBUF = 4
_CPT = 160
_PHASES = 4
_PCPT = _CPT // _PHASES
_EPT = _CPT * _CHUNK
_EPAD = _EPT * _TILES

_ROWS_PER_TILE = 632
_LAST_START = _N - _ROWS_PER_TILE


def _mm1_body(x_ref, wa_ref, wb_ref, o0_ref, o1_ref):
    xx = x_ref[...]
    o0_ref[...] = jnp.dot(xx, wa_ref[...], preferred_element_type=jnp.float32)
    o1_ref[...] = jnp.dot(xx, wb_ref[...], preferred_element_type=jnp.float32)


_matmul1 = pl.pallas_call(
    _mm1_body,
    grid=(10,),
    in_specs=[
        pl.BlockSpec((1000, _D), lambda i: (i, 0)),
        pl.BlockSpec((_D, _HALF), lambda i: (0, 0)),
        pl.BlockSpec((_D, _HALF), lambda i: (0, 0)),
    ],
    out_specs=[
        pl.BlockSpec((1000, _HALF), lambda i: (i, 0)),
        pl.BlockSpec((1000, _HALF), lambda i: (i, 0)),
    ],
    out_shape=[jax.ShapeDtypeStruct((_N, _HALF), jnp.float32)] * 2,
)


def _mm2_body(a_ref, b_ref, w2_ref, o0_ref, o1_ref):
    a = jnp.maximum(a_ref[...], 0.0)
    b = jnp.maximum(b_ref[...], 0.0)
    w = w2_ref[...]
    o0_ref[...] = (
        jnp.dot(a, w[:_HALF, :_HALF], preferred_element_type=jnp.float32)
        + jnp.dot(b, w[_HALF:, :_HALF], preferred_element_type=jnp.float32)
    )
    o1_ref[...] = (
        jnp.dot(a, w[:_HALF, _HALF:], preferred_element_type=jnp.float32)
        + jnp.dot(b, w[_HALF:, _HALF:], preferred_element_type=jnp.float32)
    )


_matmul2 = pl.pallas_call(
    _mm2_body,
    grid=(10,),
    in_specs=[
        pl.BlockSpec((1000, _HALF), lambda i: (i, 0)),
        pl.BlockSpec((1000, _HALF), lambda i: (i, 0)),
        pl.BlockSpec((_D, _D), lambda i: (0, 0)),
    ],
    out_specs=[
        pl.BlockSpec((1000, _HALF), lambda i: (i, 0)),
        pl.BlockSpec((1000, _HALF), lambda i: (i, 0)),
    ],
    out_shape=[jax.ShapeDtypeStruct((_N, _HALF), jnp.float32)] * 2,
)


def _spmm_body(src_hbm, dst_hbm, w_hbm, h0_hbm, h1_hbm, init0_hbm, init1_hbm,
               out0_hbm, out1_hbm, acc, gidx, sidx, wbuf,
               rows0, rows1, rows2, rows3,
               semg0, semg1, semg2, semg3, sems0, sems1, sems2, sems3):
    c = lax.axis_index("c")
    s = lax.axis_index("s")
    row0 = jnp.where(s == _TILES - 1, _LAST_START, s * _ROWS_PER_TILE)
    rsl = pl.ds(row0, _ROWS_PER_TILE)

    @pl.when(c == 0)
    def _():
        pltpu.sync_copy(init0_hbm.at[rsl], acc.at[rsl])

    @pl.when(c == 1)
    def _():
        pltpu.sync_copy(init1_hbm.at[rsl], acc.at[rsl])

    plsc.subcore_barrier()

    h_hbm = (h0_hbm, h1_hbm)
    rows = (rows0, rows1, rows2, rows3)
    semg = (semg0, semg1, semg2, semg3)
    sems = (sems0, sems1, sems2, sems3)

    def _gather(k, b):
        @pl.when(c == 0)
        def _():
            pltpu.async_copy(h_hbm[0].at[gidx.at[k]], rows[b], semg[b])

        @pl.when(c == 1)
        def _():
            pltpu.async_copy(h_hbm[1].at[gidx.at[k]], rows[b], semg[b])

    def _gather_wait(k, b):
        @pl.when(c == 0)
        def _():
            pltpu.make_async_copy(h_hbm[0].at[gidx.at[k]], rows[b],
                                  semg[b]).wait()

        @pl.when(c == 1)
        def _():
            pltpu.make_async_copy(h_hbm[1].at[gidx.at[k]], rows[b],
                                  semg[b]).wait()

    def _scatter(k, b):
        pltpu.async_copy(rows[b], acc.at[sidx.at[k]], sems[b], add=True)

    def _scatter_wait(k, b):
        pltpu.make_async_copy(rows[b], acc.at[sidx.at[k]], sems[b]).wait()

    def _scale(k, b):
        @plsc.parallel_loop(0, _CHUNK, unroll=4)
        def _(e):
            wv = plsc.load_gather(
                wbuf, [jnp.full((_LANES,), k, jnp.int32),
                       jnp.full((_LANES,), e, jnp.int32)])
            for j in range(_HALF // _LANES):
                sl = (e, pl.ds(j * _LANES, _LANES))
                rows[b][sl] = rows[b][sl] * wv

    @pl.loop(0, _PHASES)
    def _(p):
        psl = pl.ds(p * _PCPT, _PCPT)
        pltpu.sync_copy(src_hbm.at[s, psl], gidx)
        pltpu.sync_copy(dst_hbm.at[s, psl], sidx)
        pltpu.sync_copy(w_hbm.at[s, psl], wbuf)

        for b in range(_NBUF):
            _gather(b, b)

        @pl.loop(0, _PCPT, step=_NBUF)
        def _(k):
            for b in range(_NBUF):
                _gather_wait(k + b, b)
                _scale(k + b, b)
                _scatter(k + b, b)
                br = (b + 3) % _NBUF
                kr = k + b + 3

                @pl.when(k + b >= 1)
                def _(_br=br, _kr=kr):
                    _scatter_wait(_kr - _NBUF, _br)

                    @pl.when(jnp.logical_and(_kr >= _NBUF, _kr < _PCPT))
                    def _():
                        _gather(_kr, _br)

        _scatter_wait(_PCPT - 1, (_PCPT - 1) % _NBUF)

    plsc.subcore_barrier()

    @pl.when(c == 0)
    def _():
        pltpu.sync_copy(acc.at[rsl], out0_hbm.at[rsl])

    @pl.when(c == 1)
    def _():
        pltpu.sync_copy(acc.at[rsl], out1_hbm.at[rsl])


_sc_params = pltpu.CompilerParams()
if "needs_layout_passes" in pltpu.CompilerParams.__dataclass_fields__:
    _sc_params = dataclasses.replace(_sc_params, needs_layout_passes=False)

_spmm = pl.kernel(
    _spmm_body,
    compiler_params=_sc_params,
    out_type=(
        jax.ShapeDtypeStruct((_N, _HALF), jnp.float32),
        jax.ShapeDtypeStruct((_N, _HALF), jnp.float32),
    ),
    mesh=plsc.VectorSubcoreMesh(
        core_axis_name="c", subcore_axis_name="s", num_cores=2, num_subcores=16
    ),
    scratch_types=[
        pltpu.VMEM_SHARED((_N, _HALF), jnp.float32),
        pltpu.VMEM((_PCPT, _CHUNK), jnp.int32),
        pltpu.VMEM((_PCPT, _CHUNK), jnp.int32),
        pltpu.VMEM((_PCPT, _CHUNK), jnp.float32),
        pltpu.VMEM((_CHUNK, _HALF), jnp.float32),
        pltpu.VMEM((_CHUNK, _HALF), jnp.float32),
        pltpu.VMEM((_CHUNK, _HALF), jnp.float32),
        pltpu.VMEM((_CHUNK, _HALF), jnp.float32),
        pltpu.SemaphoreType.DMA,
        pltpu.SemaphoreType.DMA,
        pltpu.SemaphoreType.DMA,
        pltpu.SemaphoreType.DMA,
        pltpu.SemaphoreType.DMA,
        pltpu.SemaphoreType.DMA,
        pltpu.SemaphoreType.DMA,
        pltpu.SemaphoreType.DMA,
    ],
)


def kernel(x, edge_index, edge_weight, w1, b1, w2, b2):
    dst = edge_index[0]
    src = edge_index[1]
    pad = _EPAD - _E
    shape3 = (_TILES, _CPT, _CHUNK)
    src_p = jnp.concatenate([src, jnp.zeros((pad,), jnp.int32)]).reshape(shape3)
    dst_p = jnp.concatenate([dst, jnp.zeros((pad,), jnp.int32)]).reshape(shape3)
    w_p = jnp.concatenate(
        [edge_weight, jnp.zeros((pad,), jnp.float32)]).reshape(shape3)

    h0, h1 = _matmul1(x, w1[:, :_HALF], w1[:, _HALF:])
    b1_0 = jnp.broadcast_to(b1[None, :_HALF], (_N, _HALF))
    b1_1 = jnp.broadcast_to(b1[None, _HALF:], (_N, _HALF))
    s1_0, s1_1 = _spmm(src_p, dst_p, w_p, h0, h1, b1_0, b1_1)

    h2_0, h2_1 = _matmul2(s1_0, s1_1, w2)
    b2_0 = jnp.broadcast_to(b2[None, :_HALF], (_N, _HALF))
    b2_1 = jnp.broadcast_to(b2[None, _HALF:], (_N, _HALF))
    o0, o1 = _spmm(src_p, dst_p, w_p, h2_0, h2_1, b2_0, b2_1)

    return jnp.concatenate([o0, o1], axis=1)
